# baseline (speedup 1.0000x reference)
import functools

import jax
import jax.numpy as jnp
from jax import lax
from jax.experimental import pallas as pl
from jax.experimental.pallas import tpu as pltpu
from jax.experimental.pallas import tpu_sc as plsc

N = 10000
E = 320000
D = 128
NW = 32
CN = 125
CNP = 128
NCH = E // (NW * CN)
ROWS_PER_TILE = N // 16


def _prep_body(ew_ref, et_ref, p_ref, eet1_ref, num1_ref, eet2_ref, num2_ref):
    ew = ew_ref[...]
    et = et_ref[...]
    mu = jnp.mean(ew)
    var = jnp.mean((ew - mu) ** 2)
    zs = (ew - mu) * lax.rsqrt(var + 1e-5)
    for eet_ref, num_ref, off in ((eet1_ref, num1_ref, 0), (eet2_ref, num2_ref, 3)):
        ge = p_ref[0, off]
        be = p_ref[0, off + 1]
        dl = p_ref[0, off + 2]
        eet = jnp.exp(dl * et)
        eet_ref[...] = eet
        num_ref[...] = (zs * ge + be) * eet


def _prep(ew2, et2, params):
    e2 = ew2.shape
    return pl.pallas_call(
        _prep_body,
        out_shape=[jax.ShapeDtypeStruct(e2, jnp.float32)] * 4,
        in_specs=[
            pl.BlockSpec(memory_space=pltpu.VMEM),
            pl.BlockSpec(memory_space=pltpu.VMEM),
            pl.BlockSpec(memory_space=pltpu.SMEM),
        ],
        out_specs=[pl.BlockSpec(memory_space=pltpu.VMEM)] * 4,
    )(ew2, et2, params)


NB = 16
NBATCH = NCH // NB


def _agg_body(x_hbm, src_hbm, dst_hbm, num_hbm, eet_hbm,
              aggr_out, s_out,
              aggr_sh, s_sh, msrc, mdst, mnum, meet, xbuf, z2d, z1k,
              gsem, psem, esem):
    c = lax.axis_index("c")
    sid = lax.axis_index("s")
    wid = sid * 2 + c

    zv = jnp.zeros((16,), jnp.float32)

    for i in range(8):
        for k in range(D // 16):
            z2d[i, pl.ds(k * 16, 16)] = zv
    for i in range(z1k.shape[0] // 16):
        z1k[pl.ds(i * 16, 16)] = zv

    base = pl.multiple_of(sid * 624, 8)

    def _zblk(r, _):
        pltpu.sync_copy(z2d, aggr_sh.at[pl.ds(base + r * 8, 8)])
        return 0
    lax.fori_loop(0, 78, _zblk, 0)

    @pl.when(sid == 15)
    def _():
        pltpu.sync_copy(z2d, aggr_sh.at[pl.ds(9984, 8)])
        pltpu.sync_copy(z2d, aggr_sh.at[pl.ds(9992, 8)])

    @pl.when(sid == 0)
    def _():
        for i in range(N // 200):
            pltpu.sync_copy(z1k, s_sh.at[pl.ds(i * 200, 200)])

    for k in range(CNP // 16):
        msrc[0, 0, pl.ds(k * 16, 16)] = (
            jax.lax.broadcasted_iota(jnp.int32, (16,), 0) + k * 16)

    plsc.subcore_barrier()

    pltpu.async_copy(x_hbm.at[msrc.at[0, 0]], xbuf.at[0], gsem)

    def _process(slot, b):
        def _grp(g, _):
            sv = mnum[0, b, pl.ds(g * 16, 16)]
            for i in range(16):
                sc_ = sv[i]
                e = g * 16 + i
                for k in range(D // 16):
                    sl = pl.ds(k * 16, 16)
                    xbuf[slot, e, sl] = xbuf[slot, e, sl] * sc_
            return 0
        lax.fori_loop(0, CNP // 16, _grp, 0)
        pltpu.sync_copy(xbuf.at[slot], aggr_sh.at[mdst.at[0, b]], add=True)
        pltpu.async_copy(meet.at[0, b], s_sh.at[mdst.at[0, b]], esem, add=True)

    def _gwait(slot):
        pltpu.make_async_copy(x_hbm.at[msrc.at[0, 0]],
                              xbuf.at[slot], gsem).wait()

    def _batch(bb, _):
        j0 = pl.multiple_of(bb * NB, 8)
        pltpu.async_copy(src_hbm.at[wid, pl.ds(j0, NB)], msrc.at[0], psem)
        pltpu.async_copy(dst_hbm.at[wid, pl.ds(j0, NB)], mdst.at[0], psem)
        pltpu.async_copy(num_hbm.at[wid, pl.ds(j0, NB)], mnum.at[0], psem)
        cp = pltpu.async_copy(eet_hbm.at[wid, pl.ds(j0, NB)], meet.at[0], psem)
        for _ in range(4):
            cp.wait()

        _gwait(0)
        pltpu.async_copy(x_hbm.at[msrc.at[0, 0]], xbuf.at[0], gsem)

        def _pair(p, _):
            b0 = 2 * p
            b1 = b0 + 1
            pltpu.async_copy(x_hbm.at[msrc.at[0, b1]], xbuf.at[1], gsem)
            _gwait(0)
            _process(0, b0)
            nxt = lax.rem(b0 + 2, NB)
            pltpu.async_copy(x_hbm.at[msrc.at[0, nxt]], xbuf.at[0], gsem)
            _gwait(1)
            _process(1, b1)
            return 0
        lax.fori_loop(0, NB // 2, _pair, 0)

        for _ in range(NB):
            pltpu.make_async_copy(meet.at[0, 0], s_sh.at[mdst.at[0, 0]],
                                  esem).wait()
        return 0
    lax.fori_loop(0, NBATCH, _batch, 0)

    _gwait(0)

    plsc.subcore_barrier()

    pltpu.sync_copy(aggr_sh.at[pl.ds(base, 624)],
                    aggr_out.at[c, pl.ds(base, 624)])

    @pl.when(sid == 15)
    def _():
        pltpu.sync_copy(aggr_sh.at[pl.ds(9984, 16)],
                        aggr_out.at[c, pl.ds(9984, 16)])

    @pl.when(sid == 0)
    def _():
        pltpu.sync_copy(s_sh, s_out.at[c])


def _agg(x, src3, dst3, num3, eet3):
    mesh = plsc.VectorSubcoreMesh(core_axis_name="c", subcore_axis_name="s")
    call = pl.kernel(
        _agg_body,
        out_type=[
            jax.ShapeDtypeStruct((2, N, D), jnp.float32),
            jax.ShapeDtypeStruct((2, N), jnp.float32),
        ],
        mesh=mesh,
        scratch_types=[
            pltpu.VMEM_SHARED((N, D), jnp.float32),
            pltpu.VMEM_SHARED((N,), jnp.float32),
            pltpu.VMEM((1, NB, CNP), jnp.int32),
            pltpu.VMEM((1, NB, CNP), jnp.int32),
            pltpu.VMEM((1, NB, CNP), jnp.float32),
            pltpu.VMEM((1, NB, CNP), jnp.float32),
            pltpu.VMEM((2, CNP, D), jnp.float32),
            pltpu.VMEM((8, D), jnp.float32),
            pltpu.VMEM((200,), jnp.float32),
            pltpu.SemaphoreType.DMA,
            pltpu.SemaphoreType.DMA,
            pltpu.SemaphoreType.DMA,
        ],
        compiler_params=pltpu.CompilerParams(use_tc_tiling_on_sc=False),
    )
    return call(x, src3, dst3, num3, eet3)


def _dense_body(x_ref, agg_ref, s_ref, wst_ref, wht_ref, wft_ref,
                bs_ref, bh_ref, gf_ref, bf_ref, bfc_ref, out_ref, *, residual):
    xv = x_ref[...]
    agg = agg_ref[0] + agg_ref[1]
    sv = s_ref[0] + s_ref[1]
    rs = 1.0 / jnp.where(sv == 0.0, 1.0, sv)
    aggr = agg * rs
    h = (jnp.dot(xv, wst_ref[...], preferred_element_type=jnp.float32)
         + jnp.dot(aggr, wht_ref[...], preferred_element_type=jnp.float32)
         + bs_ref[...] + bh_ref[...])
    mu = jnp.mean(h, axis=0, keepdims=True)
    var = jnp.mean((h - mu) ** 2, axis=0, keepdims=True)
    hn = (h - mu) * lax.rsqrt(var + 1e-5) * gf_ref[...] + bf_ref[...]
    act = jnp.maximum(hn, 0.0)
    o = jnp.dot(act, wft_ref[...], preferred_element_type=jnp.float32) + bfc_ref[...]
    if residual:
        o = o + xv
    out_ref[...] = o


def _dense(x, aggP, sP, WsT, WhT, WfT, bs, bh, gf, bf, bfc, residual):
    body = functools.partial(_dense_body, residual=residual)
    return pl.pallas_call(
        body,
        out_shape=jax.ShapeDtypeStruct((N, D), jnp.float32),
    )(x, aggP, sP.reshape(2, N, 1), WsT, WhT, WfT,
      bs.reshape(1, D), bh.reshape(1, D), gf.reshape(1, D),
      bf.reshape(1, D), bfc.reshape(1, D))


def kernel(x, edge_index, edge_weights, edge_times, node_time,
           delta1, ge1, be1, Ws1, bs1, Wh1, bh1, gf1, bf1, Wf1, bfc1,
           delta2, ge2, be2, Ws2, bs2, Wh2, bh2, gf2, bf2, Wf2, bfc2):
    assert x.shape == (N, D) and edge_weights.shape == (E,)

    src = edge_index[0]
    dst = edge_index[1]


    params = jnp.stack([ge1, be1, delta1, ge2, be2, delta2,
                        jnp.float32(0.0), jnp.float32(0.0)]).reshape(1, 8)
    eet1, num1, eet2, num2 = _prep(
        edge_weights.reshape(E // D, D), edge_times.reshape(E // D, D), params)

    to3 = lambda a: a.reshape(NW, NCH, CN)
    pad_i = lambda a3: jnp.concatenate([a3, a3[..., :CNP - CN]], axis=-1)
    pad_z = lambda a3: jnp.concatenate(
        [a3, jnp.zeros(a3.shape[:2] + (CNP - CN,), a3.dtype)], axis=-1)
    src3 = pad_i(to3(src))
    dst3 = pad_i(to3(dst))

    agg1, s1 = _agg(x, src3, dst3, pad_z(to3(num1.reshape(E))),
                    pad_z(to3(eet1.reshape(E))))
    x2 = _dense(x, agg1, s1, Ws1.T, Wh1.T, Wf1.T, bs1, bh1, gf1, bf1, bfc1,
                residual=True)
    agg2, s2 = _agg(x2, src3, dst3, pad_z(to3(num2.reshape(E))),
                    pad_z(to3(eet2.reshape(E))))
    out = _dense(x2, agg2, s2, Ws2.T, Wh2.T, Wf2.T, bs2, bh2, gf2, bf2, bfc2,
                 residual=False)
    return out
